# baseline (speedup 1.0000x reference)
import jax
import jax.numpy as jnp
from jax.experimental import pallas as pl
from jax.experimental.pallas import tpu as pltpu

_IN = 2
_H = 32
_OUT = 3
_PACK = 4
_OMEGA = 30.0
_TR = 2048

_INV_PI = 0.31830987334251404
_PI_HI = 3.140625
_PI_MID = 0.0009676535846665502
_S1 = 1.0
_S2 = -0.16666658222675323
_S3 = 0.008333050645887852
_S4 = -0.00019809044897556305
_S5 = 2.6051632175949635e-06


def _relu_sin(z):
    n = jnp.rint(z * _INV_PI)
    r = (z - n * _PI_HI) - n * _PI_MID
    q = r * r
    p = _S4 + q * _S5
    p = _S3 + q * p
    p = _S2 + q * p
    p = _S1 + q * p
    s = r * p
    sb = jax.lax.shift_left(jnp.bitwise_and(n.astype(jnp.int32), 1), 31)
    s = jax.lax.bitcast_convert_type(
        jax.lax.bitcast_convert_type(s, jnp.int32) ^ sb, jnp.float32)
    return jnp.maximum(s, 0.0)


def _mlp_kernel(x_ref, w0_ref, b0_ref, w1_ref, b1_ref, wh_ref, bh_ref, o_ref):
    z0 = jnp.dot(x_ref[...], w0_ref[...], preferred_element_type=jnp.float32)
    h0 = _relu_sin(z0 + b0_ref[...])
    z1 = jnp.dot(h0, w1_ref[...], preferred_element_type=jnp.float32)
    h1 = _relu_sin(z1 + b1_ref[...])
    z2 = jnp.dot(h1, wh_ref[...], preferred_element_type=jnp.float32)
    o_ref[...] = z2 + bh_ref[...]


def _round_bf16(a):
    u = jax.lax.bitcast_convert_type(a.astype(jnp.float32), jnp.uint32)
    u = (u + jnp.uint32(0x7FFF) + ((u >> 16) & jnp.uint32(1))) & jnp.uint32(0xFFFF0000)
    return jax.lax.bitcast_convert_type(u, jnp.float32)


@jax.jit
def _run(x, w0, b0, w1, b1, wh, bh):
    B, S, D = x.shape
    M = B * S
    R = M // _PACK
    eye = jnp.eye(_PACK, dtype=jnp.float32)

    w0b = jnp.kron(eye, (_OMEGA * w0).astype(jnp.float32))
    b0r = jnp.tile(_round_bf16(_OMEGA * b0), (1, _PACK))
    w1b = jnp.kron(eye, (_OMEGA * w1).astype(jnp.float32))
    b1r = jnp.tile(_round_bf16(_OMEGA * b1), (1, _PACK))
    whb = jnp.kron(eye, wh.astype(jnp.float32))
    bhr = jnp.tile(_round_bf16(bh), (1, _PACK))

    x4 = x.reshape(R, _PACK * _IN)

    grid = (R // _TR,)
    out = pl.pallas_call(
        _mlp_kernel,
        out_shape=jax.ShapeDtypeStruct((R, _PACK * _OUT), jnp.float32),
        grid=grid,
        in_specs=[
            pl.BlockSpec((_TR, _PACK * _IN), lambda i: (i, 0)),
            pl.BlockSpec((_PACK * _IN, _PACK * _H), lambda i: (0, 0)),
            pl.BlockSpec((1, _PACK * _H), lambda i: (0, 0)),
            pl.BlockSpec((_PACK * _H, _PACK * _H), lambda i: (0, 0)),
            pl.BlockSpec((1, _PACK * _H), lambda i: (0, 0)),
            pl.BlockSpec((_PACK * _H, _PACK * _OUT), lambda i: (0, 0)),
            pl.BlockSpec((1, _PACK * _OUT), lambda i: (0, 0)),
        ],
        out_specs=pl.BlockSpec((_TR, _PACK * _OUT), lambda i: (i, 0)),
        compiler_params=pltpu.CompilerParams(
            dimension_semantics=("parallel",),
        ),
        cost_estimate=pl.CostEstimate(
            flops=2 * M * ((_IN + 1) * _H + (_H + 1) * _H + (_H + 1) * _OUT),
            transcendentals=0,
            bytes_accessed=(_IN + _OUT) * 4 * M,
        ),
    )(x4, w0b, b0r, w1b, b1r, whb, bhr)

    return out.reshape(B, S, _OUT)


def kernel(x, w0, b0, w1, b1, wh, bh):
    return _run(x, w0, b0, w1, b1, wh, bh)
